# Initial kernel scaffold; baseline (speedup 1.0000x reference)
import jax
import jax.numpy as jnp
from jax.experimental import pallas as pl


def kernel(x, edge_index, datanorm, W1, b1, W2, b2, W3, b3, Wc1, bc1, Wc2, bc2, We1, be1, We2, be2, We3, be3):
    raise NotImplementedError("write your pallas kernel here")


# speedup vs baseline: 3.4043x; 3.4043x over previous
import functools

import jax
import jax.numpy as jnp
from jax import lax
from jax.experimental import pallas as pl
from jax.experimental.pallas import tpu as pltpu
from jax.experimental.pallas import tpu_sc as plsc

_N = 50000
_E = 1600000
_L = 128
_NBLK = _E // _L
_NW = 32
_NS = 16
_BPW = _NBLK // _NW
_TAIL = _NBLK - _NW * _BPW
_KB = 10
_NCHUNK = _BPW // _KB
_ST = 3128


def _worker_id():
    c = lax.axis_index("c")
    s = lax.axis_index("s")
    return s * 2 + c, c, s


def _tiled_copy_rows(src, dst, s, n_rows):
    last = n_rows - 15 * _ST

    @pl.when(s < 15)
    def _():
        pltpu.sync_copy(src.at[pl.ds(s * _ST, _ST)], dst.at[pl.ds(s * _ST, _ST)])

    @pl.when(s == 15)
    def _():
        pltpu.sync_copy(src.at[pl.ds(15 * _ST, last)], dst.at[pl.ds(15 * _ST, last)])


def _sc_gather_pairs(tab, row2d, col2d):
    n, d = tab.shape
    mesh = plsc.VectorSubcoreMesh(core_axis_name="c", subcore_axis_name="s")

    def body(tab_hbm, row_hbm, col_hbm, gi_hbm, gj_hbm, tab_sh, idxr, idxc, bufi, bufj, sem):
        w, c, s = _worker_id()
        _tiled_copy_rows(tab_hbm, tab_sh, s, _N)
        plsc.subcore_barrier()

        def do_blocks(b0, nb):
            pltpu.sync_copy(row_hbm.at[pl.ds(b0, nb)], idxr.at[pl.ds(0, nb)])
            pltpu.sync_copy(col_hbm.at[pl.ds(b0, nb)], idxc.at[pl.ds(0, nb)])
            cps = []
            for j in range(nb):
                cps.append(pltpu.async_copy(
                    tab_sh.at[idxc.at[j, 0]],
                    bufi.at[pl.ds(j * _L, _L)], sem))
                cps.append(pltpu.async_copy(
                    tab_sh.at[idxr.at[j, 0]],
                    bufj.at[pl.ds(j * _L, _L)], sem))
            for cp in cps:
                cp.wait()
            pltpu.sync_copy(bufi.at[pl.ds(0, nb * _L)], gi_hbm.at[pl.ds(b0 * _L, nb * _L)])
            pltpu.sync_copy(bufj.at[pl.ds(0, nb * _L)], gj_hbm.at[pl.ds(b0 * _L, nb * _L)])

        def chunk(g, carry):
            do_blocks(w * _BPW + g * _KB, _KB)
            return carry
        lax.fori_loop(0, _NCHUNK, chunk, 0)

        @pl.when(w < _TAIL)
        def _():
            do_blocks(_NBLK - _TAIL + w, 1)

    fn = pl.kernel(
        body,
        out_type=(jax.ShapeDtypeStruct((_E, d), jnp.float32),
                  jax.ShapeDtypeStruct((_E, d), jnp.float32)),
        mesh=mesh,
        compiler_params=pltpu.CompilerParams(use_tc_tiling_on_sc=False),
        scratch_types=[
            pltpu.VMEM_SHARED((n, d), jnp.float32),
            pltpu.VMEM((_KB, 1, _L), jnp.int32),
            pltpu.VMEM((_KB, 1, _L), jnp.int32),
            pltpu.VMEM((_KB * _L, d), jnp.float32),
            pltpu.VMEM((_KB * _L, d), jnp.float32),
            pltpu.SemaphoreType.DMA,
        ],
    )
    return fn(tab, row2d, col2d)


def _sc_scatter_add(m2, col2d, zeros):
    d = m2.shape[1]
    mesh = plsc.VectorSubcoreMesh(core_axis_name="c", subcore_axis_name="s")

    def body(m2_hbm, col_hbm, z_hbm, out_hbm, acc_sh, idx, buf, sem):
        w, c, s = _worker_id()
        _tiled_copy_rows(z_hbm, acc_sh, s, _N)
        plsc.subcore_barrier()

        def do_blocks(b0, nb):
            pltpu.sync_copy(col_hbm.at[pl.ds(b0, nb)], idx.at[pl.ds(0, nb)])
            pltpu.sync_copy(m2_hbm.at[pl.ds(b0 * _L, nb * _L)],
                            buf.at[pl.ds(0, nb * _L)])
            for j in range(nb):
                pltpu.sync_copy(buf.at[pl.ds(j * _L, _L)],
                                acc_sh.at[idx.at[j, 0]], add=True)

        def chunk(g, carry):
            do_blocks(w * _BPW + g * _KB, _KB)
            return carry
        lax.fori_loop(0, _NCHUNK, chunk, 0)

        @pl.when(w < _TAIL)
        def _():
            do_blocks(_NBLK - _TAIL + w, 1)

        plsc.subcore_barrier()
        _tiled_copy_rows(acc_sh, out_hbm.at[c], s, _N)

    fn = pl.kernel(
        body,
        out_type=jax.ShapeDtypeStruct((2, _N, d), jnp.float32),
        mesh=mesh,
        compiler_params=pltpu.CompilerParams(use_tc_tiling_on_sc=False),
        scratch_types=[
            pltpu.VMEM_SHARED((_N, d), jnp.float32),
            pltpu.VMEM((_KB, 1, _L), jnp.int32),
            pltpu.VMEM((_KB * _L, d), jnp.float32),
            pltpu.SemaphoreType.DMA,
        ],
    )
    return fn(m2, col2d, zeros)


def _elu(v):
    return jnp.where(v > 0, v, jnp.exp(jnp.minimum(v, 0.0)) - 1.0)


def _node_mlp_body(x_ref, dn_ref, w1_ref, b1_ref, w2_ref, b2_ref, w3_ref, b3_ref, o_ref):
    xn = x_ref[...] * dn_ref[...]
    h = jnp.tanh(jnp.dot(xn, w1_ref[...], preferred_element_type=jnp.float32) + b1_ref[...])
    h = jnp.tanh(jnp.dot(h, w2_ref[...], preferred_element_type=jnp.float32) + b2_ref[...])
    h = jnp.tanh(jnp.dot(h, w3_ref[...], preferred_element_type=jnp.float32) + b3_ref[...])
    o_ref[...] = jnp.concatenate([h, xn], axis=1)


def _tc_node_mlp(x, datanorm, W1, b1, W2, b2, W3, b3):
    bn = 5000
    grid = (_N // bn,)
    full = lambda a: pl.BlockSpec(a.shape, lambda i: (0,) * a.ndim)
    return pl.pallas_call(
        _node_mlp_body,
        grid=grid,
        in_specs=[pl.BlockSpec((bn, 5), lambda i: (i, 0)),
                  full(datanorm), full(W1), full(b1), full(W2), full(b2),
                  full(W3), full(b3)],
        out_specs=pl.BlockSpec((bn, 21), lambda i: (i, 0)),
        out_shape=jax.ShapeDtypeStruct((_N, 21), jnp.float32),
    )(x, datanorm, W1, b1, W2, b2, W3, b3)


def _edge_mlp1_body(xi_ref, xj_ref, wa_ref, wb_ref, b1_ref, w2_ref, b2_ref, o_ref):
    m = (jnp.dot(xi_ref[...], wa_ref[...], preferred_element_type=jnp.float32)
         + jnp.dot(xj_ref[...], wb_ref[...], preferred_element_type=jnp.float32)
         + b1_ref[...])
    m = _elu(m)
    m = _elu(jnp.dot(m, w2_ref[...], preferred_element_type=jnp.float32) + b2_ref[...])
    o_ref[...] = m


def _tc_edge_mlp1(xi, xj, wa, wb, b1, w2, b2):
    be = 4000
    grid = (_E // be,)
    full = lambda a: pl.BlockSpec(a.shape, lambda i: (0,) * a.ndim)
    return pl.pallas_call(
        _edge_mlp1_body,
        grid=grid,
        in_specs=[pl.BlockSpec((be, 21), lambda i: (i, 0)),
                  pl.BlockSpec((be, 21), lambda i: (i, 0)),
                  full(wa), full(wb), full(b1), full(w2), full(b2)],
        out_specs=pl.BlockSpec((be, 16), lambda i: (i, 0)),
        out_shape=jax.ShapeDtypeStruct((_E, 16), jnp.float32),
    )(xi, xj, wa, wb, b1, w2, b2)


def _sum_partials_body(p_ref, o_ref):
    o_ref[...] = p_ref[0] + p_ref[1]


def _tc_sum_partials(partials):
    bn = 5000
    grid = (_N // bn,)
    return pl.pallas_call(
        _sum_partials_body,
        grid=grid,
        in_specs=[pl.BlockSpec((2, bn, 16), lambda i: (0, i, 0))],
        out_specs=pl.BlockSpec((bn, 16), lambda i: (i, 0)),
        out_shape=jax.ShapeDtypeStruct((_N, 16), jnp.float32),
    )(partials)


def _edge_mlp2_body(hr_ref, hc_ref, wa_ref, wb_ref, b1_ref, w2_ref, b2_ref,
                    w3_ref, b3_ref, o_ref):
    e = (jnp.dot(hr_ref[...], wa_ref[...], preferred_element_type=jnp.float32)
         + jnp.dot(hc_ref[...], wb_ref[...], preferred_element_type=jnp.float32)
         + b1_ref[...])
    e = _elu(e)
    e = _elu(jnp.dot(e, w2_ref[...], preferred_element_type=jnp.float32) + b2_ref[...])
    e = jnp.dot(e, w3_ref[...], preferred_element_type=jnp.float32) + b3_ref[...]
    m = jnp.max(e, axis=1, keepdims=True)
    sh = e - m
    lse = jnp.log(jnp.sum(jnp.exp(sh), axis=1, keepdims=True))
    o_ref[...] = sh - lse


def _tc_edge_mlp2(hr, hc, wa, wb, b1, w2, b2, w3, b3):
    be = 4000
    grid = (_E // be,)
    full = lambda a: pl.BlockSpec(a.shape, lambda i: (0,) * a.ndim)
    return pl.pallas_call(
        _edge_mlp2_body,
        grid=grid,
        in_specs=[pl.BlockSpec((be, 16), lambda i: (i, 0)),
                  pl.BlockSpec((be, 16), lambda i: (i, 0)),
                  full(wa), full(wb), full(b1), full(w2), full(b2),
                  full(w3), full(b3)],
        out_specs=pl.BlockSpec((be, 4), lambda i: (i, 0)),
        out_shape=jax.ShapeDtypeStruct((_E, 4), jnp.float32),
    )(hr, hc, wa, wb, b1, w2, b2, w3, b3)


def kernel(x, edge_index, datanorm, W1, b1, W2, b2, W3, b3,
           Wc1, bc1, Wc2, bc2, We1, be1, We2, be2, We3, be3):
    row2d = edge_index[0].reshape(_NBLK, 1, _L)
    col2d = edge_index[1].reshape(_NBLK, 1, _L)

    feat = _tc_node_mlp(x, datanorm.reshape(1, 5), W1, b1.reshape(1, -1),
                        W2, b2.reshape(1, -1), W3, b3.reshape(1, -1))

    xi, xj = _sc_gather_pairs(feat, row2d, col2d)

    wa = Wc1[:21] - Wc1[21:]
    wb = Wc1[21:]
    m2 = _tc_edge_mlp1(xi, xj, wa, wb, bc1.reshape(1, -1), Wc2, bc2.reshape(1, -1))

    partials = _sc_scatter_add(m2, col2d, jnp.zeros((_N, 16), jnp.float32))
    H = _tc_sum_partials(partials)

    hr, hc = _sc_gather_pairs(H, row2d, col2d)

    return _tc_edge_mlp2(hr, hc, We1[:16], We1[16:], be1.reshape(1, -1),
                         We2, be2.reshape(1, -1), We3, be3.reshape(1, -1))
